# baseline (speedup 1.0000x reference)
import functools

import jax
import jax.numpy as jnp
from jax import lax
from jax.experimental import pallas as pl
from jax.experimental.pallas import tpu as pltpu
from jax.experimental.pallas import tpu_sc as plsc

_CAP = 1_000_000
_D = 32
_B = 16384
_NC = 2
_NS = 16
_NW = _NC * _NS
_UPW = _B // _NW
_ELW = _UPW * _D
_ROWS = _ELW // 128
_IROWS = _UPW // 128


def _copy_body(mv_in, iw_in, mv_out, iw_out, sem0, sem1):
    c0 = pltpu.make_async_copy(mv_in, mv_out, sem0)
    c1 = pltpu.make_async_copy(iw_in, iw_out, sem1)
    c0.start()
    c1.start()
    c0.wait()
    c1.wait()


def _copy(mv_flat, iw):
    return pl.pallas_call(
        _copy_body,
        in_specs=[
            pl.BlockSpec(memory_space=pl.MemorySpace.ANY),
            pl.BlockSpec(memory_space=pl.MemorySpace.ANY),
        ],
        out_specs=[
            pl.BlockSpec(memory_space=pl.MemorySpace.ANY),
            pl.BlockSpec(memory_space=pl.MemorySpace.ANY),
        ],
        out_shape=[
            jax.ShapeDtypeStruct((_CAP * _D,), jnp.float32),
            jax.ShapeDtypeStruct((_CAP,), jnp.float32),
        ],
        scratch_shapes=[pltpu.SemaphoreType.DMA, pltpu.SemaphoreType.DMA],
    )(mv_flat, iw)


_MESH = plsc.VectorSubcoreMesh(core_axis_name="c", subcore_axis_name="s")


@functools.partial(
    pl.kernel,
    out_type=(),
    mesh=_MESH,
    scratch_types=[
        pltpu.VMEM((_ROWS, 128), jnp.int32),
        pltpu.VMEM((_ROWS, 128), jnp.float32),
        pltpu.VMEM((_ROWS, 128), jnp.float32),
        pltpu.VMEM((_IROWS, 128), jnp.int32),
        pltpu.VMEM((_IROWS, 128), jnp.float32),
        pltpu.SemaphoreType.DMA,
    ],
)
def _sc_update(mv_flat, iw, gidx_hbm, nv_hbm, ipos_hbm, mv_out, iw_out,
               gidx, vals, nvv, ipos, iwv, sem):
    wid = lax.axis_index("s") * _NC + lax.axis_index("c")

    pltpu.sync_copy(gidx_hbm.at[wid], gidx)
    pltpu.sync_copy(nv_hbm.at[wid], nvv)
    pltpu.sync_copy(ipos_hbm.at[wid], ipos)

    def _fire_gather(r, carry):
        pltpu.async_copy(mv_flat.at[gidx.at[r]], vals.at[r], sem)
        return carry

    lax.fori_loop(0, _ROWS, _fire_gather, 0)
    for r in range(_IROWS):
        pltpu.async_copy(iw.at[ipos.at[r]], iwv.at[r], sem)
    pltpu.make_async_copy(nv_hbm.at[wid], vals, sem).wait()
    pltpu.make_async_copy(nv_hbm.at[wid].at[pl.ds(0, _IROWS)], iwv, sem).wait()

    def _blend(r, carry):
        for c0 in range(0, 128, 16):
            a = vals[r, pl.ds(c0, 16)]
            b = nvv[r, pl.ds(c0, 16)]
            vals[r, pl.ds(c0, 16)] = a * 0.9 + b * 0.1
        return carry

    lax.fori_loop(0, _ROWS, _blend, 0)

    for r in range(_IROWS):
        for c0 in range(0, 128, 16):
            w = iwv[r, pl.ds(c0, 16)]
            iwv[r, pl.ds(c0, 16)] = jnp.minimum(w + 0.01, 1.0)

    def _fire_scatter(r, carry):
        pltpu.async_copy(vals.at[r], mv_out.at[gidx.at[r]], sem)
        return carry

    lax.fori_loop(0, _ROWS, _fire_scatter, 0)
    for r in range(_IROWS):
        pltpu.async_copy(iwv.at[r], iw_out.at[ipos.at[r]], sem)
    pltpu.make_async_copy(nv_hbm.at[wid], vals, sem).wait()
    pltpu.make_async_copy(nv_hbm.at[wid].at[pl.ds(0, _IROWS)], iwv, sem).wait()


def kernel(memory_values, importance_weights, position, new_value):
    pos = position.astype(jnp.int32)
    iota = lax.iota(jnp.int32, _B)
    pos_sorted, perm = lax.sort_key_val(pos, iota, is_stable=True)
    is_end = jnp.concatenate(
        [pos_sorted[1:] != pos_sorted[:-1], jnp.ones((1,), jnp.bool_)])
    end_slot = jnp.where(is_end, iota, _B)
    win_slot = lax.cummin(end_slot, axis=0, reverse=True)
    win_orig = perm[win_slot]
    wnv = jnp.zeros((_B,), jnp.int32).at[perm].set(win_orig)
    nv_eff = new_value[wnv]

    col = lax.iota(jnp.int32, _D)
    gidx = (pos[:, None] * _D + col[None, :]).reshape(_NW, _ROWS, 128)
    nv3 = nv_eff.reshape(_NW, _ROWS, 128)
    ipos3 = pos.reshape(_NW, _IROWS, 128)

    mv_flat = memory_values.reshape(-1)
    mv0, iw0 = _copy(mv_flat, importance_weights)
    mv_ref = jax.new_ref(mv0)
    iw_ref = jax.new_ref(iw0)
    _sc_update(mv_flat, importance_weights, gidx, nv3, ipos3, mv_ref, iw_ref)
    return mv_ref[...].reshape(_CAP, _D), iw_ref[...]
